# baseline (speedup 1.0000x reference)
import jax
import jax.numpy as jnp
from jax import lax
from jax.experimental import pallas as pl
from jax.experimental.pallas import tpu as pltpu
from jax.experimental.pallas import tpu_sc as plsc

EMBED_DIM = 32
BATCH = 16384

NUM_CORES = 2
NUM_SUBCORES = 16
LANES = 16
NUM_WORKERS = NUM_CORES * NUM_SUBCORES
ROWS_PER_WORKER = BATCH // NUM_WORKERS
CHUNK = 128
NUM_CHUNKS = ROWS_PER_WORKER // CHUNK
GROUPS = ROWS_PER_WORKER // LANES


def _body(user_hbm, pos_hbm, neg_hbm, uembT_hbm, iembT_hbm,
          pos_out_hbm, neg_out_hbm,
          idx_u, idx_i, idx_j, val_u, val_i, val_j,
          out_p, out_n, sem):
    wid = lax.axis_index("s") * NUM_CORES + lax.axis_index("c")
    base = wid * ROWS_PER_WORKER

    for k in range(NUM_CHUNKS):
        off = base + k * CHUNK
        pltpu.sync_copy(user_hbm.at[pl.ds(off, CHUNK)], idx_u.at[k])
        pltpu.sync_copy(pos_hbm.at[pl.ds(off, CHUNK)], idx_i.at[k])
        pltpu.sync_copy(neg_hbm.at[pl.ds(off, CHUNK)], idx_j.at[k])

    copies = []
    for d in range(EMBED_DIM):
        src_u = uembT_hbm.at[d]
        src_i = iembT_hbm.at[d]
        for k in range(NUM_CHUNKS):
            sl = pl.ds(k * CHUNK, CHUNK)
            copies.append(pltpu.async_copy(src_u.at[idx_u.at[k]], val_u.at[d, sl], sem))
            copies.append(pltpu.async_copy(src_i.at[idx_i.at[k]], val_i.at[d, sl], sem))
            copies.append(pltpu.async_copy(src_i.at[idx_j.at[k]], val_j.at[d, sl], sem))
    for c in copies:
        c.wait()

    @pl.loop(0, GROUPS)
    def _(g):
        sl = pl.ds(g * LANES, LANES)
        acc_p = jnp.zeros((LANES,), jnp.float32)
        acc_n = jnp.zeros((LANES,), jnp.float32)
        for d in range(EMBED_DIM):
            u = val_u[d, sl]
            acc_p = acc_p + u * val_i[d, sl]
            acc_n = acc_n + u * val_j[d, sl]
        out_p[sl] = acc_p
        out_n[sl] = acc_n

    pltpu.sync_copy(out_p, pos_out_hbm.at[pl.ds(base, ROWS_PER_WORKER)])
    pltpu.sync_copy(out_n, neg_out_hbm.at[pl.ds(base, ROWS_PER_WORKER)])


@jax.jit
def _scores(user, pos_item, neg_item, user_emb, item_emb):
    mesh = plsc.VectorSubcoreMesh(core_axis_name="c", subcore_axis_name="s")
    f = pl.kernel(
        _body,
        out_type=(
            jax.ShapeDtypeStruct((BATCH,), jnp.float32),
            jax.ShapeDtypeStruct((BATCH,), jnp.float32),
        ),
        mesh=mesh,
        compiler_params=pltpu.CompilerParams(
            needs_layout_passes=False, use_tc_tiling_on_sc=False),
        scratch_types=[
            pltpu.VMEM((NUM_CHUNKS, CHUNK), jnp.int32),
            pltpu.VMEM((NUM_CHUNKS, CHUNK), jnp.int32),
            pltpu.VMEM((NUM_CHUNKS, CHUNK), jnp.int32),
            pltpu.VMEM((EMBED_DIM, ROWS_PER_WORKER), jnp.float32),
            pltpu.VMEM((EMBED_DIM, ROWS_PER_WORKER), jnp.float32),
            pltpu.VMEM((EMBED_DIM, ROWS_PER_WORKER), jnp.float32),
            pltpu.VMEM((ROWS_PER_WORKER,), jnp.float32),
            pltpu.VMEM((ROWS_PER_WORKER,), jnp.float32),
            pltpu.SemaphoreType.DMA,
        ],
    )
    return f(user, pos_item, neg_item, user_emb.T, item_emb.T)


def kernel(user, pos_item, neg_item, user_emb, item_emb):
    pos_score, neg_score = _scores(user, pos_item, neg_item, user_emb, item_emb)
    return (pos_score[:, None], neg_score[:, None])


# speedup vs baseline: 5.7035x; 5.7035x over previous
import jax
import jax.numpy as jnp
from jax import lax
from jax.experimental import pallas as pl
from jax.experimental.pallas import tpu as pltpu
from jax.experimental.pallas import tpu_sc as plsc

EMBED_DIM = 32
BATCH = 16384

NUM_CORES = 2
NUM_SUBCORES = 16
LANES = 16
HALF = EMBED_DIM // LANES
NUM_WORKERS = NUM_CORES * NUM_SUBCORES
ROWS_PER_WORKER = BATCH // NUM_WORKERS
CHUNK = 128
NUM_CHUNKS = ROWS_PER_WORKER // CHUNK
GROUPS = ROWS_PER_WORKER // LANES


def _body(user_hbm, pos_hbm, neg_hbm, uemb_hbm, iemb_hbm,
          pos_out_hbm, neg_out_hbm,
          idx_u, idx_i, idx_j, rows_u, rows_i, rows_j,
          prod_p, prod_n, out_p, out_n, sem):
    wid = lax.axis_index("s") * NUM_CORES + lax.axis_index("c")
    base = wid * ROWS_PER_WORKER

    for k in range(NUM_CHUNKS):
        off = base + k * CHUNK
        pltpu.sync_copy(user_hbm.at[pl.ds(off, CHUNK)], idx_u.at[k])
        pltpu.sync_copy(pos_hbm.at[pl.ds(off, CHUNK)], idx_i.at[k])
        pltpu.sync_copy(neg_hbm.at[pl.ds(off, CHUNK)], idx_j.at[k])

    copies = []
    for k in range(NUM_CHUNKS):
        sl = pl.ds(k * CHUNK, CHUNK)
        copies.append(pltpu.async_copy(uemb_hbm.at[idx_u.at[k]], rows_u.at[sl, :], sem))
        copies.append(pltpu.async_copy(iemb_hbm.at[idx_i.at[k]], rows_i.at[sl, :], sem))
        copies.append(pltpu.async_copy(iemb_hbm.at[idx_j.at[k]], rows_j.at[sl, :], sem))
    for c in copies:
        c.wait()

    @pl.loop(0, ROWS_PER_WORKER)
    def _(b):
        u0 = rows_u[b, pl.ds(0, LANES)]
        u1 = rows_u[b, pl.ds(LANES, LANES)]
        i0 = rows_i[b, pl.ds(0, LANES)]
        i1 = rows_i[b, pl.ds(LANES, LANES)]
        j0 = rows_j[b, pl.ds(0, LANES)]
        j1 = rows_j[b, pl.ds(LANES, LANES)]
        prod_p[pl.ds(b * LANES, LANES)] = u0 * i0 + u1 * i1
        prod_n[pl.ds(b * LANES, LANES)] = u0 * j0 + u1 * j1

    lane_iota = lax.iota(jnp.int32, LANES)

    @pl.loop(0, GROUPS)
    def _(g):
        flat = (g * LANES + lane_iota) * LANES
        acc_p = plsc.load_gather(prod_p, [flat])
        acc_n = plsc.load_gather(prod_n, [flat])
        for c in range(1, LANES):
            acc_p = acc_p + plsc.load_gather(prod_p, [flat + c])
            acc_n = acc_n + plsc.load_gather(prod_n, [flat + c])
        out_p[pl.ds(g * LANES, LANES)] = acc_p
        out_n[pl.ds(g * LANES, LANES)] = acc_n

    pltpu.sync_copy(out_p, pos_out_hbm.at[pl.ds(base, ROWS_PER_WORKER)])
    pltpu.sync_copy(out_n, neg_out_hbm.at[pl.ds(base, ROWS_PER_WORKER)])


@jax.jit
def _scores(user, pos_item, neg_item, user_emb, item_emb):
    mesh = plsc.VectorSubcoreMesh(core_axis_name="c", subcore_axis_name="s")
    f = pl.kernel(
        _body,
        out_type=(
            jax.ShapeDtypeStruct((BATCH,), jnp.float32),
            jax.ShapeDtypeStruct((BATCH,), jnp.float32),
        ),
        mesh=mesh,
        compiler_params=pltpu.CompilerParams(
            needs_layout_passes=False, use_tc_tiling_on_sc=False),
        scratch_types=[
            pltpu.VMEM((NUM_CHUNKS, CHUNK), jnp.int32),
            pltpu.VMEM((NUM_CHUNKS, CHUNK), jnp.int32),
            pltpu.VMEM((NUM_CHUNKS, CHUNK), jnp.int32),
            pltpu.VMEM((ROWS_PER_WORKER, EMBED_DIM), jnp.float32),
            pltpu.VMEM((ROWS_PER_WORKER, EMBED_DIM), jnp.float32),
            pltpu.VMEM((ROWS_PER_WORKER, EMBED_DIM), jnp.float32),
            pltpu.VMEM((ROWS_PER_WORKER * LANES,), jnp.float32),
            pltpu.VMEM((ROWS_PER_WORKER * LANES,), jnp.float32),
            pltpu.VMEM((ROWS_PER_WORKER,), jnp.float32),
            pltpu.VMEM((ROWS_PER_WORKER,), jnp.float32),
            pltpu.SemaphoreType.DMA,
        ],
    )
    return f(user, pos_item, neg_item, user_emb, item_emb)


def kernel(user, pos_item, neg_item, user_emb, item_emb):
    pos_score, neg_score = _scores(user, pos_item, neg_item, user_emb, item_emb)
    return (pos_score[:, None], neg_score[:, None])
